# Initial kernel scaffold; baseline (speedup 1.0000x reference)
import jax
import jax.numpy as jnp
from jax.experimental import pallas as pl


def kernel(x, pos, batch, locW0, locb0, locW1, locb1, gloW0, glob0):
    raise NotImplementedError("write your pallas kernel here")


# speedup vs baseline: 1.0093x; 1.0093x over previous
import functools
import math

import jax
import jax.numpy as jnp
from jax.experimental import pallas as pl
from jax.experimental.pallas import tpu as pltpu

_RATIO = 0.25
_K = 32
_NUM_FREQ = 10


def _fps(pos, n_samples):
    N = pos.shape[0]
    idx0 = jnp.zeros((n_samples,), dtype=jnp.int32)
    dists0 = jnp.full((N,), jnp.inf, dtype=pos.dtype)
    d_first = jnp.sum((pos - pos[0]) ** 2, axis=-1)
    dists0 = jnp.minimum(dists0, d_first)

    def body(i, state):
        idx, dists = state
        nxt = jnp.argmax(dists).astype(jnp.int32)
        idx = idx.at[i].set(nxt)
        d = jnp.sum((pos - pos[nxt]) ** 2, axis=-1)
        dists = jnp.minimum(dists, d)
        return (idx, dists)

    idx, _ = jax.lax.fori_loop(1, n_samples, body, (idx0, dists0))
    return idx


def _knn(pos_all, pos_query, k):
    qq = jnp.sum(pos_query * pos_query, axis=-1)
    pp = jnp.sum(pos_all * pos_all, axis=-1)
    d2 = qq[:, None] + pp[None, :] - 2.0 * (pos_query @ pos_all.T)
    _, col = jax.lax.top_k(-d2, k)
    return col


def _pos_enc(coords):
    freq = (2.0 ** jnp.arange(_NUM_FREQ, dtype=coords.dtype)) * math.pi
    scaled = coords[..., None] * freq
    s = jnp.sin(scaled)
    c = jnp.cos(scaled)
    enc = jnp.stack([s, c], axis=-1).reshape(coords.shape[:-1] + (-1,))
    return jnp.concatenate([coords, enc], axis=-1)


def _final_mlp_body(h_ref, w_ref, b_ref, o_ref):
    o_ref[...] = (
        jnp.dot(h_ref[...], w_ref[...], preferred_element_type=jnp.float32)
        + b_ref[...]
    )


def _final_mlp(h, w, b):
    M = h.shape[0]
    return pl.pallas_call(
        _final_mlp_body,
        out_shape=jax.ShapeDtypeStruct((M, w.shape[1]), jnp.float32),
    )(h, w, b.reshape(1, -1))


def kernel(x, pos, batch, locW0, locb0, locW1, locb1, gloW0, glob0):
    N = pos.shape[0]
    n_samples = int(math.ceil(_RATIO * N))
    idx = _fps(pos, n_samples)
    col = _knn(pos, pos[idx], _K)
    colf = col.reshape(-1)
    row = jnp.repeat(jnp.arange(n_samples, dtype=jnp.int32), _K)
    pos_diff = pos[colf] - pos[idx][row]
    penc = _pos_enc(pos_diff)
    edge_input = jnp.concatenate([x[colf], penc], axis=-1)
    h = jax.nn.relu(edge_input @ locW0 + locb0)
    h = h @ locW1 + locb1
    seg = jnp.max(h.reshape(n_samples, _K, -1), axis=1)
    out = _final_mlp(seg, gloW0, glob0)
    return (out, pos[idx], batch[idx])


# speedup vs baseline: 4.1827x; 4.1441x over previous
import functools
import math

import jax
import jax.numpy as jnp
from jax.experimental import pallas as pl
from jax.experimental.pallas import tpu as pltpu

_RATIO = 0.25
_K = 32
_NUM_FREQ = 10


_FPS_R, _FPS_C = 80, 128
_OUT_R = 20


def _fps_body(n, n_samples, p3_ref, idx_ref, qx_ref, qy_ref, qz_ref):
    px = p3_ref[0]
    py = p3_ref[1]
    pz = p3_ref[2]
    ii = (
        jax.lax.broadcasted_iota(jnp.int32, (_FPS_R, _FPS_C), 0) * _FPS_C
        + jax.lax.broadcasted_iota(jnp.int32, (_FPS_R, _FPS_C), 1)
    )
    io = (
        jax.lax.broadcasted_iota(jnp.int32, (_OUT_R, _FPS_C), 0) * _FPS_C
        + jax.lax.broadcasted_iota(jnp.int32, (_OUT_R, _FPS_C), 1)
    )
    valid = ii < n

    def coords_at(j):
        sel = ii == j
        zero = jnp.float32(0.0)
        ax = jnp.sum(jnp.where(sel, px, zero))
        ay = jnp.sum(jnp.where(sel, py, zero))
        az = jnp.sum(jnp.where(sel, pz, zero))
        return ax, ay, az

    def dist_to(ax, ay, az):
        dx = px - ax
        dy = py - ay
        dz = pz - az
        return (dx * dx + dy * dy) + dz * dz

    ax0, ay0, az0 = coords_at(jnp.int32(0))
    d0 = dist_to(ax0, ay0, az0)
    dists0 = jnp.where(valid, d0, -jnp.inf)

    idx0 = jnp.zeros((_OUT_R, _FPS_C), jnp.int32)
    oh0 = io == 0
    qx0 = jnp.where(oh0, ax0, 0.0)
    qy0 = jnp.where(oh0, ay0, 0.0)
    qz0 = jnp.where(oh0, az0, 0.0)

    def body(i, state):
        dists, idxs, qx, qy, qz = state
        m = jnp.max(dists)
        nxt = jnp.min(jnp.where(dists == m, ii, jnp.int32(2**30)))
        ax, ay, az = coords_at(nxt)
        d = dist_to(ax, ay, az)
        dists = jnp.minimum(dists, d)
        oh = io == i
        idxs = jnp.where(oh, nxt, idxs)
        qx = jnp.where(oh, ax, qx)
        qy = jnp.where(oh, ay, qy)
        qz = jnp.where(oh, az, qz)
        return (dists, idxs, qx, qy, qz)

    _, idxs, qx, qy, qz = jax.lax.fori_loop(
        1, n_samples, body, (dists0, idx0, qx0, qy0, qz0)
    )
    idx_ref[...] = idxs
    qx_ref[...] = qx
    qy_ref[...] = qy
    qz_ref[...] = qz


def _fps(pos, n_samples):
    N = pos.shape[0]
    npad = _FPS_R * _FPS_C
    p3 = jnp.zeros((3, npad), jnp.float32)
    p3 = p3.at[:, :N].set(pos.T).reshape(3, _FPS_R, _FPS_C)
    out_shapes = [
        jax.ShapeDtypeStruct((_OUT_R, _FPS_C), jnp.int32),
        jax.ShapeDtypeStruct((_OUT_R, _FPS_C), jnp.float32),
        jax.ShapeDtypeStruct((_OUT_R, _FPS_C), jnp.float32),
        jax.ShapeDtypeStruct((_OUT_R, _FPS_C), jnp.float32),
    ]
    idxs, qx, qy, qz = pl.pallas_call(
        functools.partial(_fps_body, N, n_samples),
        out_shape=out_shapes,
    )(p3)
    idx = idxs.reshape(-1)[:n_samples]
    pos_q = jnp.stack(
        [qx.reshape(-1)[:n_samples], qy.reshape(-1)[:n_samples],
         qz.reshape(-1)[:n_samples]], axis=1)
    return idx, pos_q


def _knn(pos_all, pos_query, k):
    qq = jnp.sum(pos_query * pos_query, axis=-1)
    pp = jnp.sum(pos_all * pos_all, axis=-1)
    d2 = qq[:, None] + pp[None, :] - 2.0 * (pos_query @ pos_all.T)
    _, col = jax.lax.top_k(-d2, k)
    return col


def _pos_enc(coords):
    freq = (2.0 ** jnp.arange(_NUM_FREQ, dtype=coords.dtype)) * math.pi
    scaled = coords[..., None] * freq
    s = jnp.sin(scaled)
    c = jnp.cos(scaled)
    enc = jnp.stack([s, c], axis=-1).reshape(coords.shape[:-1] + (-1,))
    return jnp.concatenate([coords, enc], axis=-1)


def _final_mlp_body(h_ref, w_ref, b_ref, o_ref):
    o_ref[...] = (
        jnp.dot(h_ref[...], w_ref[...], preferred_element_type=jnp.float32)
        + b_ref[...]
    )


def _final_mlp(h, w, b):
    M = h.shape[0]
    return pl.pallas_call(
        _final_mlp_body,
        out_shape=jax.ShapeDtypeStruct((M, w.shape[1]), jnp.float32),
    )(h, w, b.reshape(1, -1))


def kernel(x, pos, batch, locW0, locb0, locW1, locb1, gloW0, glob0):
    N = pos.shape[0]
    n_samples = int(math.ceil(_RATIO * N))
    idx, pos_q = _fps(pos, n_samples)
    col = _knn(pos, pos_q, _K)
    colf = col.reshape(-1)
    row = jnp.repeat(jnp.arange(n_samples, dtype=jnp.int32), _K)
    pos_diff = pos[colf] - pos_q[row]
    penc = _pos_enc(pos_diff)
    edge_input = jnp.concatenate([x[colf], penc], axis=-1)
    h = jax.nn.relu(edge_input @ locW0 + locb0)
    h = h @ locW1 + locb1
    seg = jnp.max(h.reshape(n_samples, _K, -1), axis=1)
    out = _final_mlp(seg, gloW0, glob0)
    return (out, pos_q, batch[idx])


# speedup vs baseline: 4.4369x; 1.0608x over previous
import functools
import math

import jax
import jax.numpy as jnp
from jax import lax
from jax.experimental import pallas as pl
from jax.experimental.pallas import tpu as pltpu
from jax.experimental.pallas import tpu_sc as plsc

_RATIO = 0.25
_K = 32
_NUM_FREQ = 10


_FPS_R, _FPS_C = 80, 128
_OUT_R = 20


def _fps_body(n, n_samples, p3_ref, idx_ref, qx_ref, qy_ref, qz_ref):
    px = p3_ref[0]
    py = p3_ref[1]
    pz = p3_ref[2]
    ii = (
        jax.lax.broadcasted_iota(jnp.int32, (_FPS_R, _FPS_C), 0) * _FPS_C
        + jax.lax.broadcasted_iota(jnp.int32, (_FPS_R, _FPS_C), 1)
    )
    io = (
        jax.lax.broadcasted_iota(jnp.int32, (_OUT_R, _FPS_C), 0) * _FPS_C
        + jax.lax.broadcasted_iota(jnp.int32, (_OUT_R, _FPS_C), 1)
    )
    valid = ii < n

    def coords_at(j):
        sel = ii == j
        zero = jnp.float32(0.0)
        ax = jnp.sum(jnp.where(sel, px, zero))
        ay = jnp.sum(jnp.where(sel, py, zero))
        az = jnp.sum(jnp.where(sel, pz, zero))
        return ax, ay, az

    def dist_to(ax, ay, az):
        dx = px - ax
        dy = py - ay
        dz = pz - az
        return (dx * dx + dy * dy) + dz * dz

    ax0, ay0, az0 = coords_at(jnp.int32(0))
    d0 = dist_to(ax0, ay0, az0)
    dists0 = jnp.where(valid, d0, -jnp.inf)

    idx0 = jnp.zeros((_OUT_R, _FPS_C), jnp.int32)
    oh0 = io == 0
    qx0 = jnp.where(oh0, ax0, 0.0)
    qy0 = jnp.where(oh0, ay0, 0.0)
    qz0 = jnp.where(oh0, az0, 0.0)

    def body(i, state):
        dists, idxs, qx, qy, qz = state
        m = jnp.max(dists)
        nxt = jnp.min(jnp.where(dists == m, ii, jnp.int32(2**30)))
        ax, ay, az = coords_at(nxt)
        d = dist_to(ax, ay, az)
        dists = jnp.minimum(dists, d)
        oh = io == i
        idxs = jnp.where(oh, nxt, idxs)
        qx = jnp.where(oh, ax, qx)
        qy = jnp.where(oh, ay, qy)
        qz = jnp.where(oh, az, qz)
        return (dists, idxs, qx, qy, qz)

    _, idxs, qx, qy, qz = jax.lax.fori_loop(
        1, n_samples, body, (dists0, idx0, qx0, qy0, qz0)
    )
    idx_ref[...] = idxs
    qx_ref[...] = qx
    qy_ref[...] = qy
    qz_ref[...] = qz


def _fps(pos, n_samples):
    N = pos.shape[0]
    npad = _FPS_R * _FPS_C
    p3 = jnp.zeros((3, npad), jnp.float32)
    p3 = p3.at[:, :N].set(pos.T).reshape(3, _FPS_R, _FPS_C)
    out_shapes = [
        jax.ShapeDtypeStruct((_OUT_R, _FPS_C), jnp.int32),
        jax.ShapeDtypeStruct((_OUT_R, _FPS_C), jnp.float32),
        jax.ShapeDtypeStruct((_OUT_R, _FPS_C), jnp.float32),
        jax.ShapeDtypeStruct((_OUT_R, _FPS_C), jnp.float32),
    ]
    idxs, qx, qy, qz = pl.pallas_call(
        functools.partial(_fps_body, N, n_samples),
        out_shape=out_shapes,
    )(p3)
    idx = idxs.reshape(-1)[:n_samples]
    pos_qp = jnp.stack([qx.reshape(-1), qy.reshape(-1), qz.reshape(-1)], axis=1)
    return idx, pos_qp


def _knn(pos_all, pos_query, k):
    qq = jnp.sum(pos_query * pos_query, axis=-1)
    pp = jnp.sum(pos_all * pos_all, axis=-1)
    d2 = qq[:, None] + pp[None, :] - 2.0 * (pos_query @ pos_all.T)
    _, col = jax.lax.top_k(-d2, k)
    return col


def _pos_enc(coords):
    freq = (2.0 ** jnp.arange(_NUM_FREQ, dtype=coords.dtype)) * math.pi
    scaled = coords[..., None] * freq
    s = jnp.sin(scaled)
    c = jnp.cos(scaled)
    enc = jnp.stack([s, c], axis=-1).reshape(coords.shape[:-1] + (-1,))
    return jnp.concatenate([coords, enc], axis=-1)


def _xw_body(x_ref, w_ref, o_ref):
    xw = jnp.dot(x_ref[...], w_ref[...], preferred_element_type=jnp.float32)
    o_ref[...] = jnp.concatenate(
        [xw, jnp.zeros((xw.shape[0], 64), jnp.float32)], axis=1)


_NC, _NS = 2, 16
_NW = _NC * _NS
_BQ = 2560
_BE = _BQ * _K
_BPW = _BE // _NW
_CH = 128
_NCH = _BPW // _CH


def _sc_gather_body(tab_hbm, col_hbm, xg_hbm, idx_v, row_v, sem):
    wid = lax.axis_index("s") * _NC + lax.axis_index("c")
    base = wid * _BPW

    def chunk(ci, carry):
        off = base + ci * _CH
        pltpu.sync_copy(col_hbm.at[pl.ds(off, _CH)], idx_v)
        pltpu.async_copy(tab_hbm.at[idx_v], row_v, sem).wait()
        pltpu.sync_copy(row_v, xg_hbm.at[pl.ds(off, _CH)])
        return carry

    lax.fori_loop(0, _NCH, chunk, 0)


def _sc_gather(tab, col):
    mesh = plsc.VectorSubcoreMesh(core_axis_name="c", subcore_axis_name="s",
                                  num_cores=_NC, num_subcores=_NS)
    f = pl.kernel(
        _sc_gather_body,
        out_type=jax.ShapeDtypeStruct((_BE, 128), jnp.float32),
        mesh=mesh,
        scratch_types=[
            pltpu.VMEM((_CH,), jnp.int32),
            pltpu.VMEM((_CH, 128), jnp.float32),
            pltpu.SemaphoreType.DMA,
        ],
        compiler_params=pltpu.CompilerParams(use_tc_tiling_on_sc=True),
    )
    return f(tab, col)


_QB = 128
_EB = _QB * _K


def _edge_body(xgp_ref, qe_ref, em_ref, ws_ref, wc_ref, wd_ref,
               b0_ref, w1_ref, b1_ref, wg_ref, bg_ref, out_ref):
    f32 = jnp.float32
    xgp = xgp_ref[...]
    xg = lax.slice(xgp, (0, 0), (_EB, 64))
    pg = lax.slice(xgp, (0, 64), (_EB, 80))
    pd = pg - qe_ref[...]
    scaled = jnp.dot(pd, em_ref[...], preferred_element_type=f32,
                     precision=lax.Precision.HIGHEST)
    s = jnp.sin(scaled)
    c = jnp.cos(scaled)
    h = (xg
         + jnp.dot(s, ws_ref[...], preferred_element_type=f32)
         + jnp.dot(c, wc_ref[...], preferred_element_type=f32)
         + jnp.dot(pd, wd_ref[...], preferred_element_type=f32,
                   precision=lax.Precision.HIGHEST)
         + b0_ref[...])
    h = jnp.maximum(h, 0.0)
    h = jnp.dot(h, w1_ref[...], preferred_element_type=f32) + b1_ref[...]
    seg = jnp.max(h.reshape(_QB, _K, 64), axis=1)
    out_ref[...] = (jnp.dot(seg, wg_ref[...], preferred_element_type=f32)
                    + bg_ref[...])


def _edge_pipeline(xgp, qe, em, ws, wc, wd, b0, w1, b1, wg, bg):
    grid = _BQ // _QB
    rep = lambda i: (0, 0)
    return pl.pallas_call(
        _edge_body,
        grid=(grid,),
        in_specs=[
            pl.BlockSpec((_EB, 128), lambda i: (i, 0)),
            pl.BlockSpec((_EB, 16), lambda i: (i, 0)),
            pl.BlockSpec((16, 32), rep),
            pl.BlockSpec((32, 64), rep),
            pl.BlockSpec((32, 64), rep),
            pl.BlockSpec((16, 64), rep),
            pl.BlockSpec((1, 64), rep),
            pl.BlockSpec((64, 64), rep),
            pl.BlockSpec((1, 64), rep),
            pl.BlockSpec((64, 128), rep),
            pl.BlockSpec((1, 128), rep),
        ],
        out_specs=pl.BlockSpec((_QB, 128), lambda i: (i, 0)),
        out_shape=jax.ShapeDtypeStruct((_BQ, 128), jnp.float32),
    )(xgp, qe, em, ws, wc, wd, b0, w1, b1, wg, bg)


def kernel(x, pos, batch, locW0, locb0, locW1, locb1, gloW0, glob0):
    N = pos.shape[0]
    n_samples = int(math.ceil(_RATIO * N))
    idx, pos_qp = _fps(pos, n_samples)
    pos_q = pos_qp[:n_samples]
    col = _knn(pos, pos_q, _K)

    colp = jnp.zeros((_BQ, _K), jnp.int32).at[:n_samples].set(
        col.astype(jnp.int32)).reshape(-1)

    xpad = jnp.zeros((_FPS_R * _FPS_C, x.shape[1]), jnp.float32).at[:N].set(x)
    tab = pl.pallas_call(
        _xw_body,
        out_shape=jax.ShapeDtypeStruct((_FPS_R * _FPS_C, 128), jnp.float32),
    )(xpad, locW0[: x.shape[1]])
    tab = tab.at[:N, 64:67].set(pos)

    q16 = jnp.zeros((_BQ, 16), jnp.float32).at[:, :3].set(pos_qp)
    qe16 = jnp.broadcast_to(q16[:, None, :], (_BQ, _K, 16)).reshape(_BE, 16)

    xgp = _sc_gather(tab, colp)

    D = x.shape[1]
    freq = (2.0 ** jnp.arange(_NUM_FREQ, dtype=jnp.float32)) * math.pi
    em = jnp.zeros((16, 32), jnp.float32)
    for j in range(3):
        em = em.at[j, j * _NUM_FREQ:(j + 1) * _NUM_FREQ].set(freq)
    wenc = locW0[D + 3:]
    wenc3 = wenc.reshape(3, _NUM_FREQ, 2, 64)
    ws = jnp.zeros((32, 64), jnp.float32).at[:30].set(
        wenc3[:, :, 0, :].reshape(30, 64))
    wc = jnp.zeros((32, 64), jnp.float32).at[:30].set(
        wenc3[:, :, 1, :].reshape(30, 64))
    wd = jnp.zeros((16, 64), jnp.float32).at[:3].set(locW0[D:D + 3])

    outp = _edge_pipeline(
        xgp, qe16, em, ws, wc, wd,
        locb0.reshape(1, 64), locW1, locb1.reshape(1, 64),
        gloW0, glob0.reshape(1, 128))
    return (outp[:n_samples], pos_q, batch[idx])


# speedup vs baseline: 8.1590x; 1.8389x over previous
import functools
import math

import jax
import jax.numpy as jnp
from jax import lax
from jax.experimental import pallas as pl
from jax.experimental.pallas import tpu as pltpu
from jax.experimental.pallas import tpu_sc as plsc

_RATIO = 0.25
_K = 32
_NUM_FREQ = 10


_FPS_R, _FPS_C = 80, 128
_OUT_R = 20


def _fps_body(n, n_samples, p3_ref, idx_ref, qx_ref, qy_ref, qz_ref):
    px = p3_ref[0]
    py = p3_ref[1]
    pz = p3_ref[2]
    ii = (
        jax.lax.broadcasted_iota(jnp.int32, (_FPS_R, _FPS_C), 0) * _FPS_C
        + jax.lax.broadcasted_iota(jnp.int32, (_FPS_R, _FPS_C), 1)
    )
    io = (
        jax.lax.broadcasted_iota(jnp.int32, (_OUT_R, _FPS_C), 0) * _FPS_C
        + jax.lax.broadcasted_iota(jnp.int32, (_OUT_R, _FPS_C), 1)
    )
    valid = ii < n

    def coords_at(j):
        sel = ii == j
        zero = jnp.float32(0.0)
        ax = jnp.sum(jnp.where(sel, px, zero))
        ay = jnp.sum(jnp.where(sel, py, zero))
        az = jnp.sum(jnp.where(sel, pz, zero))
        return ax, ay, az

    def dist_to(ax, ay, az):
        dx = px - ax
        dy = py - ay
        dz = pz - az
        return (dx * dx + dy * dy) + dz * dz

    ax0, ay0, az0 = coords_at(jnp.int32(0))
    d0 = dist_to(ax0, ay0, az0)
    dists0 = jnp.where(valid, d0, -jnp.inf)

    idx0 = jnp.zeros((_OUT_R, _FPS_C), jnp.int32)
    oh0 = io == 0
    qx0 = jnp.where(oh0, ax0, 0.0)
    qy0 = jnp.where(oh0, ay0, 0.0)
    qz0 = jnp.where(oh0, az0, 0.0)

    def body(i, state):
        dists, idxs, qx, qy, qz = state
        m = jnp.max(dists)
        nxt = jnp.min(jnp.where(dists == m, ii, jnp.int32(2**30)))
        ax, ay, az = coords_at(nxt)
        d = dist_to(ax, ay, az)
        dists = jnp.minimum(dists, d)
        oh = io == i
        idxs = jnp.where(oh, nxt, idxs)
        qx = jnp.where(oh, ax, qx)
        qy = jnp.where(oh, ay, qy)
        qz = jnp.where(oh, az, qz)
        return (dists, idxs, qx, qy, qz)

    _, idxs, qx, qy, qz = jax.lax.fori_loop(
        1, n_samples, body, (dists0, idx0, qx0, qy0, qz0)
    )
    idx_ref[...] = idxs
    qx_ref[...] = qx
    qy_ref[...] = qy
    qz_ref[...] = qz


def _fps(pos, n_samples):
    N = pos.shape[0]
    npad = _FPS_R * _FPS_C
    p3 = jnp.zeros((3, npad), jnp.float32)
    p3 = p3.at[:, :N].set(pos.T).reshape(3, _FPS_R, _FPS_C)
    out_shapes = [
        jax.ShapeDtypeStruct((_OUT_R, _FPS_C), jnp.int32),
        jax.ShapeDtypeStruct((_OUT_R, _FPS_C), jnp.float32),
        jax.ShapeDtypeStruct((_OUT_R, _FPS_C), jnp.float32),
        jax.ShapeDtypeStruct((_OUT_R, _FPS_C), jnp.float32),
    ]
    idxs, qx, qy, qz = pl.pallas_call(
        functools.partial(_fps_body, N, n_samples),
        out_shape=out_shapes,
    )(p3)
    idx = idxs.reshape(-1)[:n_samples]
    pos_qp = jnp.stack([qx.reshape(-1), qy.reshape(-1), qz.reshape(-1)], axis=1)
    return idx, pos_qp


_KQB = 128


def _knn_body(n, p4_ref, qx_ref, qy_ref, qz_ref, col_ref, d2_ref):
    f32 = jnp.float32
    p4 = p4_ref[...]
    qx = qx_ref[0]
    qy = qy_ref[0]
    qz = qz_ref[0]
    q4 = jnp.concatenate(
        [qx, qy, qz, jnp.zeros((1, _KQB), f32)], axis=0)
    pp = jnp.sum(p4 * p4, axis=1, keepdims=True)
    qq = qx * qx + qy * qy + qz * qz
    mm = jnp.dot(p4, q4, preferred_element_type=f32)
    d2_ref[...] = pp + qq - 2.0 * mm

    npd = p4.shape[0]
    ci = jax.lax.broadcasted_iota(jnp.int32, (npd, _KQB), 0)
    jo = jax.lax.broadcasted_iota(jnp.int32, (_K, _KQB), 0)

    def it(j, colacc):
        d2 = d2_ref[...]
        m = jnp.min(d2, axis=0, keepdims=True)
        sel = d2 == m
        idxq = jnp.min(jnp.where(sel, ci, jnp.int32(2 ** 30)),
                       axis=0, keepdims=True)
        d2_ref[...] = jnp.where(ci == idxq, jnp.float32(jnp.inf), d2)
        return jnp.where(jo == j, idxq, colacc)

    colacc = jax.lax.fori_loop(
        0, _K, it, jnp.zeros((_K, _KQB), jnp.int32))
    col_ref[...] = colacc


def _knn(pos, pos_qp, n):
    npd = _FPS_R * _FPS_C
    p4 = jnp.full((npd, 4), 1e18, jnp.float32)
    p4 = p4.at[:n, :3].set(pos).at[:, 3].set(0.0)
    qx = pos_qp[:, 0].reshape(_OUT_R, 1, _FPS_C)
    qy = pos_qp[:, 1].reshape(_OUT_R, 1, _FPS_C)
    qz = pos_qp[:, 2].reshape(_OUT_R, 1, _FPS_C)
    rep = lambda i: (0, 0)
    colb = pl.pallas_call(
        functools.partial(_knn_body, n),
        grid=(_OUT_R,),
        in_specs=[
            pl.BlockSpec((npd, 4), rep),
            pl.BlockSpec((1, 1, _FPS_C), lambda i: (i, 0, 0)),
            pl.BlockSpec((1, 1, _FPS_C), lambda i: (i, 0, 0)),
            pl.BlockSpec((1, 1, _FPS_C), lambda i: (i, 0, 0)),
        ],
        out_specs=pl.BlockSpec((_K, _KQB), lambda i: (i, 0)),
        out_shape=jax.ShapeDtypeStruct((_OUT_R * _K, _KQB), jnp.int32),
        scratch_shapes=[pltpu.VMEM((npd, _KQB), jnp.float32)],
    )(p4, qx, qy, qz)
    col = colb.reshape(_OUT_R, _K, _KQB).transpose(0, 2, 1).reshape(_BQ, _K)
    return col


def _pos_enc(coords):
    freq = (2.0 ** jnp.arange(_NUM_FREQ, dtype=coords.dtype)) * math.pi
    scaled = coords[..., None] * freq
    s = jnp.sin(scaled)
    c = jnp.cos(scaled)
    enc = jnp.stack([s, c], axis=-1).reshape(coords.shape[:-1] + (-1,))
    return jnp.concatenate([coords, enc], axis=-1)


def _xw_body(x_ref, w_ref, o_ref):
    xw = jnp.dot(x_ref[...], w_ref[...], preferred_element_type=jnp.float32)
    o_ref[...] = jnp.concatenate(
        [xw, jnp.zeros((xw.shape[0], 64), jnp.float32)], axis=1)


_NC, _NS = 2, 16
_NW = _NC * _NS
_BQ = 2560
_BE = _BQ * _K
_BPW = _BE // _NW
_CH = 128
_NCH = _BPW // _CH


def _sc_gather_body(tab_hbm, col_hbm, xg_hbm, idx_v, row_v, sem):
    wid = lax.axis_index("s") * _NC + lax.axis_index("c")
    base = wid * _BPW

    def chunk(ci, carry):
        off = base + ci * _CH
        pltpu.sync_copy(col_hbm.at[pl.ds(off, _CH)], idx_v)
        pltpu.async_copy(tab_hbm.at[idx_v], row_v, sem).wait()
        pltpu.sync_copy(row_v, xg_hbm.at[pl.ds(off, _CH)])
        return carry

    lax.fori_loop(0, _NCH, chunk, 0)


def _sc_gather(tab, col):
    mesh = plsc.VectorSubcoreMesh(core_axis_name="c", subcore_axis_name="s",
                                  num_cores=_NC, num_subcores=_NS)
    f = pl.kernel(
        _sc_gather_body,
        out_type=jax.ShapeDtypeStruct((_BE, 128), jnp.float32),
        mesh=mesh,
        scratch_types=[
            pltpu.VMEM((_CH,), jnp.int32),
            pltpu.VMEM((_CH, 128), jnp.float32),
            pltpu.SemaphoreType.DMA,
        ],
        compiler_params=pltpu.CompilerParams(use_tc_tiling_on_sc=True),
    )
    return f(tab, col)


_QB = 128
_EB = _QB * _K


def _edge_body(xgp_ref, qe_ref, em_ref, ws_ref, wc_ref, wd_ref,
               b0_ref, w1_ref, b1_ref, wg_ref, bg_ref, out_ref):
    f32 = jnp.float32
    xgp = xgp_ref[...]
    xg = lax.slice(xgp, (0, 0), (_EB, 64))
    pg = lax.slice(xgp, (0, 64), (_EB, 80))
    pd = pg - qe_ref[...]
    scaled = jnp.dot(pd, em_ref[...], preferred_element_type=f32,
                     precision=lax.Precision.HIGHEST)
    s = jnp.sin(scaled)
    c = jnp.cos(scaled)
    h = (xg
         + jnp.dot(s, ws_ref[...], preferred_element_type=f32)
         + jnp.dot(c, wc_ref[...], preferred_element_type=f32)
         + jnp.dot(pd, wd_ref[...], preferred_element_type=f32,
                   precision=lax.Precision.HIGHEST)
         + b0_ref[...])
    h = jnp.maximum(h, 0.0)
    h = jnp.dot(h, w1_ref[...], preferred_element_type=f32) + b1_ref[...]
    seg = jnp.max(h.reshape(_QB, _K, 64), axis=1)
    out_ref[...] = (jnp.dot(seg, wg_ref[...], preferred_element_type=f32)
                    + bg_ref[...])


def _edge_pipeline(xgp, qe, em, ws, wc, wd, b0, w1, b1, wg, bg):
    grid = _BQ // _QB
    rep = lambda i: (0, 0)
    return pl.pallas_call(
        _edge_body,
        grid=(grid,),
        in_specs=[
            pl.BlockSpec((_EB, 128), lambda i: (i, 0)),
            pl.BlockSpec((_EB, 16), lambda i: (i, 0)),
            pl.BlockSpec((16, 32), rep),
            pl.BlockSpec((32, 64), rep),
            pl.BlockSpec((32, 64), rep),
            pl.BlockSpec((16, 64), rep),
            pl.BlockSpec((1, 64), rep),
            pl.BlockSpec((64, 64), rep),
            pl.BlockSpec((1, 64), rep),
            pl.BlockSpec((64, 128), rep),
            pl.BlockSpec((1, 128), rep),
        ],
        out_specs=pl.BlockSpec((_QB, 128), lambda i: (i, 0)),
        out_shape=jax.ShapeDtypeStruct((_BQ, 128), jnp.float32),
    )(xgp, qe, em, ws, wc, wd, b0, w1, b1, wg, bg)


def kernel(x, pos, batch, locW0, locb0, locW1, locb1, gloW0, glob0):
    N = pos.shape[0]
    n_samples = int(math.ceil(_RATIO * N))
    idx, pos_qp = _fps(pos, n_samples)
    pos_q = pos_qp[:n_samples]
    col = _knn(pos, pos_qp, N)
    colp = col.reshape(-1)

    xpad = jnp.zeros((_FPS_R * _FPS_C, x.shape[1]), jnp.float32).at[:N].set(x)
    tab = pl.pallas_call(
        _xw_body,
        out_shape=jax.ShapeDtypeStruct((_FPS_R * _FPS_C, 128), jnp.float32),
    )(xpad, locW0[: x.shape[1]])
    tab = tab.at[:N, 64:67].set(pos)

    q16 = jnp.zeros((_BQ, 16), jnp.float32).at[:, :3].set(pos_qp)
    qe16 = jnp.broadcast_to(q16[:, None, :], (_BQ, _K, 16)).reshape(_BE, 16)

    xgp = _sc_gather(tab, colp)

    D = x.shape[1]
    freq = (2.0 ** jnp.arange(_NUM_FREQ, dtype=jnp.float32)) * math.pi
    em = jnp.zeros((16, 32), jnp.float32)
    for j in range(3):
        em = em.at[j, j * _NUM_FREQ:(j + 1) * _NUM_FREQ].set(freq)
    wenc = locW0[D + 3:]
    wenc3 = wenc.reshape(3, _NUM_FREQ, 2, 64)
    ws = jnp.zeros((32, 64), jnp.float32).at[:30].set(
        wenc3[:, :, 0, :].reshape(30, 64))
    wc = jnp.zeros((32, 64), jnp.float32).at[:30].set(
        wenc3[:, :, 1, :].reshape(30, 64))
    wd = jnp.zeros((16, 64), jnp.float32).at[:3].set(locW0[D:D + 3])

    outp = _edge_pipeline(
        xgp, qe16, em, ws, wc, wd,
        locb0.reshape(1, 64), locW1, locb1.reshape(1, 64),
        gloW0, glob0.reshape(1, 128))
    return (outp[:n_samples], pos_q, batch[idx])


# speedup vs baseline: 9.9869x; 1.2240x over previous
import functools
import math

import jax
import jax.numpy as jnp
from jax import lax
from jax.experimental import pallas as pl
from jax.experimental.pallas import tpu as pltpu
from jax.experimental.pallas import tpu_sc as plsc

_RATIO = 0.25
_K = 32
_NUM_FREQ = 10


_FPS_R, _FPS_C = 80, 128
_OUT_R = 20


def _fps_body(n, n_samples, p3_ref, pr_ref, idx_ref, qx_ref, qy_ref, qz_ref):
    px = p3_ref[0]
    py = p3_ref[1]
    pz = p3_ref[2]
    ii = (
        jax.lax.broadcasted_iota(jnp.int32, (_FPS_R, _FPS_C), 0) * _FPS_C
        + jax.lax.broadcasted_iota(jnp.int32, (_FPS_R, _FPS_C), 1)
    )
    io = (
        jax.lax.broadcasted_iota(jnp.int32, (_OUT_R, _FPS_C), 0) * _FPS_C
        + jax.lax.broadcasted_iota(jnp.int32, (_OUT_R, _FPS_C), 1)
    )
    valid = ii < n

    def coords_at(j):
        row = pr_ref[j]
        return row[0], row[1], row[2]

    def dist_to(ax, ay, az):
        dx = px - ax
        dy = py - ay
        dz = pz - az
        return (dx * dx + dy * dy) + dz * dz

    ax0, ay0, az0 = coords_at(jnp.int32(0))
    d0 = dist_to(ax0, ay0, az0)
    dists0 = jnp.where(valid, d0, -jnp.inf)

    idx0 = jnp.zeros((_OUT_R, _FPS_C), jnp.int32)
    oh0 = io == 0
    qx0 = jnp.where(oh0, ax0, 0.0)
    qy0 = jnp.where(oh0, ay0, 0.0)
    qz0 = jnp.where(oh0, az0, 0.0)

    def body(i, state):
        dists, idxs, qx, qy, qz = state
        m = jnp.max(dists)
        nxt = jnp.min(jnp.where(dists == m, ii, jnp.int32(2**30)))
        ax, ay, az = coords_at(nxt)
        d = dist_to(ax, ay, az)
        dists = jnp.minimum(dists, d)
        oh = io == i
        idxs = jnp.where(oh, nxt, idxs)
        qx = jnp.where(oh, ax, qx)
        qy = jnp.where(oh, ay, qy)
        qz = jnp.where(oh, az, qz)
        return (dists, idxs, qx, qy, qz)

    _, idxs, qx, qy, qz = jax.lax.fori_loop(
        1, n_samples, body, (dists0, idx0, qx0, qy0, qz0)
    )
    idx_ref[...] = idxs
    qx_ref[...] = qx
    qy_ref[...] = qy
    qz_ref[...] = qz


def _fps(pos, n_samples):
    N = pos.shape[0]
    npad = _FPS_R * _FPS_C
    p3 = jnp.zeros((3, npad), jnp.float32)
    p3 = p3.at[:, :N].set(pos.T).reshape(3, _FPS_R, _FPS_C)
    pr = jnp.zeros((npad, 8), jnp.float32).at[:N, :3].set(pos)
    out_shapes = [
        jax.ShapeDtypeStruct((_OUT_R, _FPS_C), jnp.int32),
        jax.ShapeDtypeStruct((_OUT_R, _FPS_C), jnp.float32),
        jax.ShapeDtypeStruct((_OUT_R, _FPS_C), jnp.float32),
        jax.ShapeDtypeStruct((_OUT_R, _FPS_C), jnp.float32),
    ]
    idxs, qx, qy, qz = pl.pallas_call(
        functools.partial(_fps_body, N, n_samples),
        out_shape=out_shapes,
    )(p3, pr)
    idx = idxs.reshape(-1)[:n_samples]
    pos_qp = jnp.stack([qx.reshape(-1), qy.reshape(-1), qz.reshape(-1)], axis=1)
    return idx, pos_qp


_KQB = 128


def _knn_body(n, p4_ref, qx_ref, qy_ref, qz_ref, col_ref, d2_ref):
    f32 = jnp.float32
    p4 = p4_ref[...]
    qx = qx_ref[0]
    qy = qy_ref[0]
    qz = qz_ref[0]
    q4 = jnp.concatenate(
        [qx, qy, qz, jnp.zeros((1, _KQB), f32)], axis=0)
    pp = jnp.sum(p4 * p4, axis=1, keepdims=True)
    qq = qx * qx + qy * qy + qz * qz
    mm = jnp.dot(p4, q4, preferred_element_type=f32)
    d2_ref[...] = pp + qq - 2.0 * mm

    npd = p4.shape[0]
    ci = jax.lax.broadcasted_iota(jnp.int32, (npd, _KQB), 0)
    jo = jax.lax.broadcasted_iota(jnp.int32, (_K, _KQB), 0)

    def it(j, colacc):
        d2 = d2_ref[...]
        idxq = jnp.argmin(d2, axis=0).astype(jnp.int32)[None, :]
        d2_ref[...] = jnp.where(ci == idxq, jnp.float32(jnp.inf), d2)
        return jnp.where(jo == j, idxq, colacc)

    colacc = jax.lax.fori_loop(
        0, _K, it, jnp.zeros((_K, _KQB), jnp.int32))
    col_ref[...] = colacc


def _knn(pos, pos_qp, n):
    npd = _FPS_R * _FPS_C
    p4 = jnp.full((npd, 4), 1e18, jnp.float32)
    p4 = p4.at[:n, :3].set(pos).at[:, 3].set(0.0)
    qx = pos_qp[:, 0].reshape(_OUT_R, 1, _FPS_C)
    qy = pos_qp[:, 1].reshape(_OUT_R, 1, _FPS_C)
    qz = pos_qp[:, 2].reshape(_OUT_R, 1, _FPS_C)
    rep = lambda i: (0, 0)
    colb = pl.pallas_call(
        functools.partial(_knn_body, n),
        grid=(_OUT_R,),
        in_specs=[
            pl.BlockSpec((npd, 4), rep),
            pl.BlockSpec((1, 1, _FPS_C), lambda i: (i, 0, 0)),
            pl.BlockSpec((1, 1, _FPS_C), lambda i: (i, 0, 0)),
            pl.BlockSpec((1, 1, _FPS_C), lambda i: (i, 0, 0)),
        ],
        out_specs=pl.BlockSpec((_K, _KQB), lambda i: (i, 0)),
        out_shape=jax.ShapeDtypeStruct((_OUT_R * _K, _KQB), jnp.int32),
        scratch_shapes=[pltpu.VMEM((npd, _KQB), jnp.float32)],
    )(p4, qx, qy, qz)
    col = colb.reshape(_OUT_R, _K, _KQB).transpose(0, 2, 1).reshape(_BQ, _K)
    return col


def _pos_enc(coords):
    freq = (2.0 ** jnp.arange(_NUM_FREQ, dtype=coords.dtype)) * math.pi
    scaled = coords[..., None] * freq
    s = jnp.sin(scaled)
    c = jnp.cos(scaled)
    enc = jnp.stack([s, c], axis=-1).reshape(coords.shape[:-1] + (-1,))
    return jnp.concatenate([coords, enc], axis=-1)


def _xw_body(x_ref, w_ref, o_ref):
    xw = jnp.dot(x_ref[...], w_ref[...], preferred_element_type=jnp.float32)
    o_ref[...] = jnp.concatenate(
        [xw, jnp.zeros((xw.shape[0], 64), jnp.float32)], axis=1)


_NC, _NS = 2, 16
_NW = _NC * _NS
_BQ = 2560
_BE = _BQ * _K
_BPW = _BE // _NW
_CH = 128
_NCH = _BPW // _CH


def _sc_gather_body(tab_hbm, col_hbm, xg_hbm, idx_v, row_v, sem):
    wid = lax.axis_index("s") * _NC + lax.axis_index("c")
    base = wid * _BPW

    def chunk(ci, carry):
        off = base + ci * _CH
        pltpu.sync_copy(col_hbm.at[pl.ds(off, _CH)], idx_v)
        pltpu.async_copy(tab_hbm.at[idx_v], row_v, sem).wait()
        pltpu.sync_copy(row_v, xg_hbm.at[pl.ds(off, _CH)])
        return carry

    lax.fori_loop(0, _NCH, chunk, 0)


def _sc_gather(tab, col):
    mesh = plsc.VectorSubcoreMesh(core_axis_name="c", subcore_axis_name="s",
                                  num_cores=_NC, num_subcores=_NS)
    f = pl.kernel(
        _sc_gather_body,
        out_type=jax.ShapeDtypeStruct((_BE, 128), jnp.float32),
        mesh=mesh,
        scratch_types=[
            pltpu.VMEM((_CH,), jnp.int32),
            pltpu.VMEM((_CH, 128), jnp.float32),
            pltpu.SemaphoreType.DMA,
        ],
        compiler_params=pltpu.CompilerParams(use_tc_tiling_on_sc=True),
    )
    return f(tab, col)


_QB = 128
_EB = _QB * _K


def _edge_body(xgp_ref, qe_ref, em_ref, ws_ref, wc_ref, wd_ref,
               b0_ref, w1_ref, b1_ref, wg_ref, bg_ref, out_ref):
    f32 = jnp.float32
    xgp = xgp_ref[...]
    xg = lax.slice(xgp, (0, 0), (_EB, 64))
    pg = lax.slice(xgp, (0, 64), (_EB, 80))
    pd = pg - qe_ref[...]
    scaled = jnp.dot(pd, em_ref[...], preferred_element_type=f32,
                     precision=lax.Precision.HIGHEST)
    s = jnp.sin(scaled)
    c = jnp.cos(scaled)
    h = (xg
         + jnp.dot(s, ws_ref[...], preferred_element_type=f32)
         + jnp.dot(c, wc_ref[...], preferred_element_type=f32)
         + jnp.dot(pd, wd_ref[...], preferred_element_type=f32,
                   precision=lax.Precision.HIGHEST)
         + b0_ref[...])
    h = jnp.maximum(h, 0.0)
    h = jnp.dot(h, w1_ref[...], preferred_element_type=f32) + b1_ref[...]
    seg = jnp.max(h.reshape(_QB, _K, 64), axis=1)
    out_ref[...] = (jnp.dot(seg, wg_ref[...], preferred_element_type=f32)
                    + bg_ref[...])


def _edge_pipeline(xgp, qe, em, ws, wc, wd, b0, w1, b1, wg, bg):
    grid = _BQ // _QB
    rep = lambda i: (0, 0)
    return pl.pallas_call(
        _edge_body,
        grid=(grid,),
        in_specs=[
            pl.BlockSpec((_EB, 128), lambda i: (i, 0)),
            pl.BlockSpec((_EB, 16), lambda i: (i, 0)),
            pl.BlockSpec((16, 32), rep),
            pl.BlockSpec((32, 64), rep),
            pl.BlockSpec((32, 64), rep),
            pl.BlockSpec((16, 64), rep),
            pl.BlockSpec((1, 64), rep),
            pl.BlockSpec((64, 64), rep),
            pl.BlockSpec((1, 64), rep),
            pl.BlockSpec((64, 128), rep),
            pl.BlockSpec((1, 128), rep),
        ],
        out_specs=pl.BlockSpec((_QB, 128), lambda i: (i, 0)),
        out_shape=jax.ShapeDtypeStruct((_BQ, 128), jnp.float32),
    )(xgp, qe, em, ws, wc, wd, b0, w1, b1, wg, bg)


def kernel(x, pos, batch, locW0, locb0, locW1, locb1, gloW0, glob0):
    N = pos.shape[0]
    n_samples = int(math.ceil(_RATIO * N))
    idx, pos_qp = _fps(pos, n_samples)
    pos_q = pos_qp[:n_samples]
    col = _knn(pos, pos_qp, N)
    colp = col.reshape(-1)

    xpad = jnp.zeros((_FPS_R * _FPS_C, x.shape[1]), jnp.float32).at[:N].set(x)
    tab = pl.pallas_call(
        _xw_body,
        out_shape=jax.ShapeDtypeStruct((_FPS_R * _FPS_C, 128), jnp.float32),
    )(xpad, locW0[: x.shape[1]])
    tab = tab.at[:N, 64:67].set(pos)

    q16 = jnp.zeros((_BQ, 16), jnp.float32).at[:, :3].set(pos_qp)
    qe16 = jnp.broadcast_to(q16[:, None, :], (_BQ, _K, 16)).reshape(_BE, 16)

    xgp = _sc_gather(tab, colp)

    D = x.shape[1]
    freq = (2.0 ** jnp.arange(_NUM_FREQ, dtype=jnp.float32)) * math.pi
    em = jnp.zeros((16, 32), jnp.float32)
    for j in range(3):
        em = em.at[j, j * _NUM_FREQ:(j + 1) * _NUM_FREQ].set(freq)
    wenc = locW0[D + 3:]
    wenc3 = wenc.reshape(3, _NUM_FREQ, 2, 64)
    ws = jnp.zeros((32, 64), jnp.float32).at[:30].set(
        wenc3[:, :, 0, :].reshape(30, 64))
    wc = jnp.zeros((32, 64), jnp.float32).at[:30].set(
        wenc3[:, :, 1, :].reshape(30, 64))
    wd = jnp.zeros((16, 64), jnp.float32).at[:3].set(locW0[D:D + 3])

    outp = _edge_pipeline(
        xgp, qe16, em, ws, wc, wd,
        locb0.reshape(1, 64), locW1, locb1.reshape(1, 64),
        gloW0, glob0.reshape(1, 128))
    return (outp[:n_samples], pos_q, batch[idx])


# speedup vs baseline: 10.5326x; 1.0546x over previous
import functools
import math

import jax
import jax.numpy as jnp
from jax import lax
from jax.experimental import pallas as pl
from jax.experimental.pallas import tpu as pltpu
from jax.experimental.pallas import tpu_sc as plsc

_RATIO = 0.25
_K = 32
_NUM_FREQ = 10


_FPS_R, _FPS_C = 80, 128
_OUT_R = 20


def _fps_body(n, n_samples, p3_ref, pr_ref, idx_ref, qx_ref, qy_ref, qz_ref):
    px = p3_ref[0]
    py = p3_ref[1]
    pz = p3_ref[2]
    ii = (
        jax.lax.broadcasted_iota(jnp.int32, (_FPS_R, _FPS_C), 0) * _FPS_C
        + jax.lax.broadcasted_iota(jnp.int32, (_FPS_R, _FPS_C), 1)
    )
    io = (
        jax.lax.broadcasted_iota(jnp.int32, (_OUT_R, _FPS_C), 0) * _FPS_C
        + jax.lax.broadcasted_iota(jnp.int32, (_OUT_R, _FPS_C), 1)
    )
    valid = ii < n

    def coords_at(j):
        row = pr_ref[j]
        return row[0], row[1], row[2]

    def dist_to(ax, ay, az):
        dx = px - ax
        dy = py - ay
        dz = pz - az
        return (dx * dx + dy * dy) + dz * dz

    ax0, ay0, az0 = coords_at(jnp.int32(0))
    d0 = dist_to(ax0, ay0, az0)
    dists0 = jnp.where(valid, d0, -jnp.inf)

    idx0 = jnp.zeros((_OUT_R, _FPS_C), jnp.int32)
    oh0 = io == 0
    qx0 = jnp.where(oh0, ax0, 0.0)
    qy0 = jnp.where(oh0, ay0, 0.0)
    qz0 = jnp.where(oh0, az0, 0.0)

    def body(i, state):
        dists, idxs, qx, qy, qz = state
        nxt = jnp.argmax(dists.reshape(-1)).astype(jnp.int32)
        ax, ay, az = coords_at(nxt)
        d = dist_to(ax, ay, az)
        dists = jnp.minimum(dists, d)
        oh = io == i
        idxs = jnp.where(oh, nxt, idxs)
        qx = jnp.where(oh, ax, qx)
        qy = jnp.where(oh, ay, qy)
        qz = jnp.where(oh, az, qz)
        return (dists, idxs, qx, qy, qz)

    _, idxs, qx, qy, qz = jax.lax.fori_loop(
        1, n_samples, body, (dists0, idx0, qx0, qy0, qz0)
    )
    idx_ref[...] = idxs
    qx_ref[...] = qx
    qy_ref[...] = qy
    qz_ref[...] = qz


def _fps(pos, n_samples):
    N = pos.shape[0]
    npad = _FPS_R * _FPS_C
    p3 = jnp.zeros((3, npad), jnp.float32)
    p3 = p3.at[:, :N].set(pos.T).reshape(3, _FPS_R, _FPS_C)
    pr = jnp.zeros((npad, 8), jnp.float32).at[:N, :3].set(pos)
    out_shapes = [
        jax.ShapeDtypeStruct((_OUT_R, _FPS_C), jnp.int32),
        jax.ShapeDtypeStruct((_OUT_R, _FPS_C), jnp.float32),
        jax.ShapeDtypeStruct((_OUT_R, _FPS_C), jnp.float32),
        jax.ShapeDtypeStruct((_OUT_R, _FPS_C), jnp.float32),
    ]
    idxs, qx, qy, qz = pl.pallas_call(
        functools.partial(_fps_body, N, n_samples),
        out_shape=out_shapes,
    )(p3, pr)
    idx = idxs.reshape(-1)[:n_samples]
    pos_qp = jnp.stack([qx.reshape(-1), qy.reshape(-1), qz.reshape(-1)], axis=1)
    return idx, pos_qp


_KQB = 128


def _knn_body(n, p4_ref, qx_ref, qy_ref, qz_ref, col_ref, d2_ref):
    f32 = jnp.float32
    p4 = p4_ref[...]
    qx = qx_ref[0]
    qy = qy_ref[0]
    qz = qz_ref[0]
    q4 = jnp.concatenate(
        [qx, qy, qz, jnp.zeros((1, _KQB), f32)], axis=0)
    pp = jnp.sum(p4 * p4, axis=1, keepdims=True)
    qq = qx * qx + qy * qy + qz * qz
    mm = jnp.dot(p4, q4, preferred_element_type=f32)
    d2_ref[...] = pp + qq - 2.0 * mm

    npd = p4.shape[0]
    ci = jax.lax.broadcasted_iota(jnp.int32, (npd, _KQB), 0)
    jo = jax.lax.broadcasted_iota(jnp.int32, (_K, _KQB), 0)

    def it(j, colacc):
        d2 = d2_ref[...]
        idxq = jnp.argmin(d2, axis=0).astype(jnp.int32)[None, :]
        d2_ref[...] = jnp.where(ci == idxq, jnp.float32(jnp.inf), d2)
        return jnp.where(jo == j, idxq, colacc)

    colacc = jax.lax.fori_loop(
        0, _K, it, jnp.zeros((_K, _KQB), jnp.int32))
    col_ref[...] = colacc


def _knn(pos, pos_qp, n):
    npd = _FPS_R * _FPS_C
    p4 = jnp.full((npd, 4), 1e18, jnp.float32)
    p4 = p4.at[:n, :3].set(pos).at[:, 3].set(0.0)
    qx = pos_qp[:, 0].reshape(_OUT_R, 1, _FPS_C)
    qy = pos_qp[:, 1].reshape(_OUT_R, 1, _FPS_C)
    qz = pos_qp[:, 2].reshape(_OUT_R, 1, _FPS_C)
    rep = lambda i: (0, 0)
    colb = pl.pallas_call(
        functools.partial(_knn_body, n),
        grid=(_OUT_R,),
        in_specs=[
            pl.BlockSpec((npd, 4), rep),
            pl.BlockSpec((1, 1, _FPS_C), lambda i: (i, 0, 0)),
            pl.BlockSpec((1, 1, _FPS_C), lambda i: (i, 0, 0)),
            pl.BlockSpec((1, 1, _FPS_C), lambda i: (i, 0, 0)),
        ],
        out_specs=pl.BlockSpec((_K, _KQB), lambda i: (i, 0)),
        out_shape=jax.ShapeDtypeStruct((_OUT_R * _K, _KQB), jnp.int32),
        scratch_shapes=[pltpu.VMEM((npd, _KQB), jnp.float32)],
    )(p4, qx, qy, qz)
    col = colb.reshape(_OUT_R, _K, _KQB).transpose(0, 2, 1).reshape(_BQ, _K)
    return col


def _pos_enc(coords):
    freq = (2.0 ** jnp.arange(_NUM_FREQ, dtype=coords.dtype)) * math.pi
    scaled = coords[..., None] * freq
    s = jnp.sin(scaled)
    c = jnp.cos(scaled)
    enc = jnp.stack([s, c], axis=-1).reshape(coords.shape[:-1] + (-1,))
    return jnp.concatenate([coords, enc], axis=-1)


def _xw_body(x_ref, w_ref, o_ref):
    xw = jnp.dot(x_ref[...], w_ref[...], preferred_element_type=jnp.float32)
    o_ref[...] = jnp.concatenate(
        [xw, jnp.zeros((xw.shape[0], 64), jnp.float32)], axis=1)


_NC, _NS = 2, 16
_NW = _NC * _NS
_BQ = 2560
_BE = _BQ * _K
_BPW = _BE // _NW
_CH = 128
_NCH = _BPW // _CH


def _sc_gather_body(tab_hbm, col_hbm, xg_hbm, idx_v, row_v, sem):
    wid = lax.axis_index("s") * _NC + lax.axis_index("c")
    base = wid * _BPW

    def chunk(ci, carry):
        off = base + ci * _CH
        pltpu.sync_copy(col_hbm.at[pl.ds(off, _CH)], idx_v)
        pltpu.async_copy(tab_hbm.at[idx_v], row_v, sem).wait()
        pltpu.sync_copy(row_v, xg_hbm.at[pl.ds(off, _CH)])
        return carry

    lax.fori_loop(0, _NCH, chunk, 0)


def _sc_gather(tab, col):
    mesh = plsc.VectorSubcoreMesh(core_axis_name="c", subcore_axis_name="s",
                                  num_cores=_NC, num_subcores=_NS)
    f = pl.kernel(
        _sc_gather_body,
        out_type=jax.ShapeDtypeStruct((_BE, 128), jnp.float32),
        mesh=mesh,
        scratch_types=[
            pltpu.VMEM((_CH,), jnp.int32),
            pltpu.VMEM((_CH, 128), jnp.float32),
            pltpu.SemaphoreType.DMA,
        ],
        compiler_params=pltpu.CompilerParams(use_tc_tiling_on_sc=True),
    )
    return f(tab, col)


_QB = 128
_EB = _QB * _K


def _edge_body(xgp_ref, qe_ref, em_ref, ws_ref, wc_ref, wd_ref,
               b0_ref, w1_ref, b1_ref, wg_ref, bg_ref, out_ref):
    f32 = jnp.float32
    xgp = xgp_ref[...]
    xg = lax.slice(xgp, (0, 0), (_EB, 64))
    pg = lax.slice(xgp, (0, 64), (_EB, 80))
    pd = pg - qe_ref[...]
    scaled = jnp.dot(pd, em_ref[...], preferred_element_type=f32,
                     precision=lax.Precision.HIGHEST)
    s = jnp.sin(scaled)
    c = jnp.cos(scaled)
    h = (xg
         + jnp.dot(s, ws_ref[...], preferred_element_type=f32)
         + jnp.dot(c, wc_ref[...], preferred_element_type=f32)
         + jnp.dot(pd, wd_ref[...], preferred_element_type=f32,
                   precision=lax.Precision.HIGHEST)
         + b0_ref[...])
    h = jnp.maximum(h, 0.0)
    h = jnp.dot(h, w1_ref[...], preferred_element_type=f32) + b1_ref[...]
    seg = jnp.max(h.reshape(_QB, _K, 64), axis=1)
    out_ref[...] = (jnp.dot(seg, wg_ref[...], preferred_element_type=f32)
                    + bg_ref[...])


def _edge_pipeline(xgp, qe, em, ws, wc, wd, b0, w1, b1, wg, bg):
    grid = _BQ // _QB
    rep = lambda i: (0, 0)
    return pl.pallas_call(
        _edge_body,
        grid=(grid,),
        in_specs=[
            pl.BlockSpec((_EB, 128), lambda i: (i, 0)),
            pl.BlockSpec((_EB, 16), lambda i: (i, 0)),
            pl.BlockSpec((16, 32), rep),
            pl.BlockSpec((32, 64), rep),
            pl.BlockSpec((32, 64), rep),
            pl.BlockSpec((16, 64), rep),
            pl.BlockSpec((1, 64), rep),
            pl.BlockSpec((64, 64), rep),
            pl.BlockSpec((1, 64), rep),
            pl.BlockSpec((64, 128), rep),
            pl.BlockSpec((1, 128), rep),
        ],
        out_specs=pl.BlockSpec((_QB, 128), lambda i: (i, 0)),
        out_shape=jax.ShapeDtypeStruct((_BQ, 128), jnp.float32),
    )(xgp, qe, em, ws, wc, wd, b0, w1, b1, wg, bg)


def kernel(x, pos, batch, locW0, locb0, locW1, locb1, gloW0, glob0):
    N = pos.shape[0]
    n_samples = int(math.ceil(_RATIO * N))
    idx, pos_qp = _fps(pos, n_samples)
    pos_q = pos_qp[:n_samples]
    col = _knn(pos, pos_qp, N)
    colp = col.reshape(-1)

    xpad = jnp.zeros((_FPS_R * _FPS_C, x.shape[1]), jnp.float32).at[:N].set(x)
    tab = pl.pallas_call(
        _xw_body,
        out_shape=jax.ShapeDtypeStruct((_FPS_R * _FPS_C, 128), jnp.float32),
    )(xpad, locW0[: x.shape[1]])
    tab = tab.at[:N, 64:67].set(pos)

    q16 = jnp.zeros((_BQ, 16), jnp.float32).at[:, :3].set(pos_qp)
    qe16 = jnp.broadcast_to(q16[:, None, :], (_BQ, _K, 16)).reshape(_BE, 16)

    xgp = _sc_gather(tab, colp)

    D = x.shape[1]
    freq = (2.0 ** jnp.arange(_NUM_FREQ, dtype=jnp.float32)) * math.pi
    em = jnp.zeros((16, 32), jnp.float32)
    for j in range(3):
        em = em.at[j, j * _NUM_FREQ:(j + 1) * _NUM_FREQ].set(freq)
    wenc = locW0[D + 3:]
    wenc3 = wenc.reshape(3, _NUM_FREQ, 2, 64)
    ws = jnp.zeros((32, 64), jnp.float32).at[:30].set(
        wenc3[:, :, 0, :].reshape(30, 64))
    wc = jnp.zeros((32, 64), jnp.float32).at[:30].set(
        wenc3[:, :, 1, :].reshape(30, 64))
    wd = jnp.zeros((16, 64), jnp.float32).at[:3].set(locW0[D:D + 3])

    outp = _edge_pipeline(
        xgp, qe16, em, ws, wc, wd,
        locb0.reshape(1, 64), locW1, locb1.reshape(1, 64),
        gloW0, glob0.reshape(1, 128))
    return (outp[:n_samples], pos_q, batch[idx])
